# Initial kernel scaffold; baseline (speedup 1.0000x reference)
import jax
import jax.numpy as jnp
from jax.experimental import pallas as pl


def kernel(x, edge_index, gnn_weight1, gnn_weight2, W_prej, alpha1):
    raise NotImplementedError("write your pallas kernel here")


# speedup vs baseline: 19.3987x; 19.3987x over previous
import functools

import jax
import jax.numpy as jnp
from jax import lax
from jax.experimental import pallas as pl
from jax.experimental.pallas import tpu as pltpu
from jax.experimental.pallas import tpu_sc as plsc

N = 10000
E = 320000
D = 128
NP = 10240
NW = 32
NB = 125
B = 80
SPAN = NP // 16

_mesh = plsc.VectorSubcoreMesh(core_axis_name="c", subcore_axis_name="s")


DW = 16


@functools.partial(
    pl.kernel,
    out_type=jax.ShapeDtypeStruct((2, NP, DW), jnp.float32),
    mesh=_mesh,
    scratch_types=[
        pltpu.VMEM((NB, B), jnp.int32),
        pltpu.VMEM((B, DW), jnp.float32),
        pltpu.VMEM_SHARED((NP, DW), jnp.float32),
    ],
)
def _deg_kernel(col_hbm, ones_hbm, zero_hbm, out_hbm, colv, onesv, sdeg):
    c = lax.axis_index("c")
    s = lax.axis_index("s")
    wid = c * 16 + s
    pltpu.sync_copy(zero_hbm, sdeg.at[pl.ds(s * SPAN, SPAN), :])
    pltpu.sync_copy(ones_hbm, onesv)
    pltpu.sync_copy(col_hbm.at[wid], colv)
    plsc.subcore_barrier()

    def step(j, carry):
        pltpu.sync_copy(onesv, sdeg.at[colv.at[j], :], add=True)
        return carry

    lax.fori_loop(0, NB, step, 0)
    plsc.subcore_barrier()
    pltpu.sync_copy(sdeg.at[pl.ds(s * SPAN, SPAN), :],
                    out_hbm.at[c, pl.ds(s * SPAN, SPAN), :])


@functools.partial(
    pl.kernel,
    out_type=jax.ShapeDtypeStruct((2, NP, D), jnp.float32),
    mesh=_mesh,
    scratch_types=[
        pltpu.VMEM((NB, B), jnp.int32),
        pltpu.VMEM((NB, B), jnp.int32),
        pltpu.VMEM((B, D), jnp.float32),
        pltpu.VMEM_SHARED((NP, D), jnp.float32),
        pltpu.SemaphoreType.DMA,
    ],
)
def _scatter_kernel(row_hbm, col_hbm, z_hbm, zero_hbm, out_hbm,
                    rowv, colv, buf, sacc, sem):
    c = lax.axis_index("c")
    s = lax.axis_index("s")
    wid = c * 16 + s
    pltpu.sync_copy(zero_hbm, sacc.at[pl.ds(s * SPAN, SPAN), :])
    pltpu.sync_copy(row_hbm.at[wid], rowv)
    pltpu.sync_copy(col_hbm.at[wid], colv)
    plsc.subcore_barrier()

    def step(j, carry):
        pltpu.async_copy(z_hbm.at[rowv.at[j]], buf, sem).wait()
        pltpu.sync_copy(buf, sacc.at[colv.at[j]], add=True)
        return carry

    lax.fori_loop(0, NB, step, 0)
    plsc.subcore_barrier()
    pltpu.sync_copy(sacc.at[pl.ds(s * SPAN, SPAN), :],
                    out_hbm.at[c, pl.ds(s * SPAN, SPAN), :])


_RB = 1000


def _dis_block(degp_ref):
    return lax.rsqrt(degp_ref[0, :, 0:1] + degp_ref[1, :, 0:1] + 1.0)


def _mm1_body(x_ref, w1_ref, wpt_ref, degp_ref, z1_ref, x0_ref):
    xb = x_ref[...]
    dis = _dis_block(degp_ref)
    z1_ref[...] = dis * jnp.dot(xb, w1_ref[...],
                                preferred_element_type=jnp.float32)
    x0_ref[...] = jnp.dot(xb, wpt_ref[...],
                          preferred_element_type=jnp.float32)


def _mm1_call(x, w1, wpt, degp):
    return pl.pallas_call(
        _mm1_body,
        grid=(N // _RB,),
        in_specs=[
            pl.BlockSpec((_RB, D), lambda i: (i, 0)),
            pl.BlockSpec((D, D), lambda i: (0, 0)),
            pl.BlockSpec((D, D), lambda i: (0, 0)),
            pl.BlockSpec((2, _RB, DW), lambda i: (0, i, 0)),
        ],
        out_specs=[
            pl.BlockSpec((_RB, D), lambda i: (i, 0)),
            pl.BlockSpec((_RB, D), lambda i: (i, 0)),
        ],
        out_shape=[
            jax.ShapeDtypeStruct((N, D), jnp.float32),
            jax.ShapeDtypeStruct((N, D), jnp.float32),
        ],
    )(x, w1, wpt, degp)


def _mid_body(s_ref, z1_ref, degp_ref, w2_ref, z2_ref):
    dis = _dis_block(degp_ref)
    h1 = jnp.maximum(dis * (s_ref[0] + s_ref[1] + z1_ref[...]), 0.0)
    z2_ref[...] = dis * jnp.dot(h1, w2_ref[...],
                                preferred_element_type=jnp.float32)


def _mid_call(s1, z1, degp, w2):
    return pl.pallas_call(
        _mid_body,
        grid=(N // _RB,),
        in_specs=[
            pl.BlockSpec((2, _RB, D), lambda i: (0, i, 0)),
            pl.BlockSpec((_RB, D), lambda i: (i, 0)),
            pl.BlockSpec((2, _RB, DW), lambda i: (0, i, 0)),
            pl.BlockSpec((D, D), lambda i: (0, 0)),
        ],
        out_specs=pl.BlockSpec((_RB, D), lambda i: (i, 0)),
        out_shape=jax.ShapeDtypeStruct((N, D), jnp.float32),
    )(s1, z1, degp, w2)


def _fin_body(s_ref, z2_ref, degp_ref, x0_ref, a_ref, o_ref):
    a = a_ref[0, 0]
    dis = _dis_block(degp_ref)
    h2 = dis * (s_ref[0] + s_ref[1] + z2_ref[...])
    X = a * h2 + (1.0 - a) * x0_ref[...]
    m = jnp.max(X, axis=1, keepdims=True)
    lse = jnp.log(jnp.sum(jnp.exp(X - m), axis=1, keepdims=True)) + m
    o_ref[...] = X - lse


def _fin_call(s2, z2, degp, x0, a11):
    return pl.pallas_call(
        _fin_body,
        grid=(N // _RB,),
        in_specs=[
            pl.BlockSpec((2, _RB, D), lambda i: (0, i, 0)),
            pl.BlockSpec((_RB, D), lambda i: (i, 0)),
            pl.BlockSpec((2, _RB, DW), lambda i: (0, i, 0)),
            pl.BlockSpec((_RB, D), lambda i: (i, 0)),
            pl.BlockSpec((1, 1), lambda i: (0, 0)),
        ],
        out_specs=pl.BlockSpec((_RB, D), lambda i: (i, 0)),
        out_shape=jax.ShapeDtypeStruct((N, D), jnp.float32),
    )(s2, z2, degp, x0, a11)


def kernel(x, edge_index, gnn_weight1, gnn_weight2, W_prej, alpha1):
    ei = edge_index.astype(jnp.int32)
    row3 = ei[0].reshape(NW, NB, B)
    col3 = ei[1].reshape(NW, NB, B)
    ones_b = jnp.ones((B, DW), jnp.float32)
    zeros_col = jnp.zeros((SPAN, DW), jnp.float32)
    zeros_slab = jnp.zeros((SPAN, D), jnp.float32)
    wpt = W_prej.T
    a11 = alpha1.reshape(1, 1).astype(jnp.float32)

    degp = _deg_kernel(col3, ones_b, zeros_col)
    z1, x0 = _mm1_call(x, gnn_weight1, wpt, degp)
    s1 = _scatter_kernel(row3, col3, z1, zeros_slab)
    z2 = _mid_call(s1, z1, degp, gnn_weight2)
    s2 = _scatter_kernel(row3, col3, z2, zeros_slab)
    return _fin_call(s2, z2, degp, x0, a11)
